# baseline (speedup 1.0000x reference)
import functools

import jax
import jax.numpy as jnp
from jax import lax
from jax.experimental import pallas as pl
from jax.experimental.pallas import tpu as pltpu
from jax.experimental.pallas import tpu_sc as plsc

N = 10000
E = 320000
NUM_CLASSES = 4
F = 4
NC = 2
NS = 16
NW = NC * NS
EPT = E // NW
WIN = 128
NWIN = 80
EPTP = NWIN * WIN
PADE = EPTP - EPT
DUMP = 2048
PADN = N + DUMP
NSET = 5

_mesh = plsc.VectorSubcoreMesh(core_axis_name="c", subcore_axis_name="s")
_sc_params = pltpu.CompilerParams(use_tc_tiling_on_sc=False)
_sc_params_nl = pltpu.CompilerParams(use_tc_tiling_on_sc=False,
                                     needs_layout_passes=False)


@functools.partial(
    pl.kernel,
    out_type=jax.ShapeDtypeStruct((NC, N), jnp.float32),
    mesh=_mesh,
    compiler_params=_sc_params,
    scratch_types=[
        pltpu.VMEM((NWIN, WIN), jnp.int32),
        pltpu.VMEM((WIN,), jnp.float32),
        pltpu.VMEM((PADN,), jnp.float32),
        pltpu.VMEM_SHARED((PADN,), jnp.float32),
        pltpu.SemaphoreType.DMA,
    ],
)
def _deg_kernel(dst_hbm, zeros_hbm, out_hbm, dst_v, ones_v, stage_v, acc_sh,
                ssem):
    c = lax.axis_index("c")
    s = lax.axis_index("s")
    wid = s * NC + c

    @pl.when(s == 0)
    def _():
        pltpu.sync_copy(zeros_hbm, stage_v)
        pltpu.sync_copy(stage_v, acc_sh)

    for i in range(WIN // 16):
        ones_v[pl.ds(i * 16, 16)] = jnp.full((16,), 1.0, jnp.float32)
    pltpu.sync_copy(dst_hbm.at[wid], dst_v)
    plsc.subcore_barrier()

    def body(w, carry):
        pltpu.async_copy(ones_v, acc_sh.at[dst_v.at[w]], ssem, add=True)
        return carry

    lax.fori_loop(0, NWIN, body, 0)

    def drain(w, carry):
        pltpu.make_async_copy(ones_v, acc_sh.at[dst_v.at[w]], ssem).wait()
        return carry

    lax.fori_loop(0, NWIN, drain, 0)
    plsc.subcore_barrier()

    @pl.when(s == 0)
    def _():
        pltpu.sync_copy(acc_sh.at[pl.ds(0, N)], stage_v.at[pl.ds(0, N)])
        pltpu.sync_copy(stage_v.at[pl.ds(0, N)], out_hbm.at[c])


def _make_prop_kernel(f):
    @functools.partial(
        pl.kernel,
        out_type=jax.ShapeDtypeStruct((NC, f, N), jnp.float32),
        mesh=_mesh,
        compiler_params=_sc_params_nl,
        scratch_types=(
            [pltpu.VMEM((NWIN, WIN), jnp.int32),
             pltpu.VMEM((NWIN, WIN), jnp.int32),
             pltpu.VMEM((NSET * f * WIN,), jnp.float32),
             pltpu.VMEM((PADN,), jnp.float32)]
            + [pltpu.VMEM((N,), jnp.float32)] * f
            + [pltpu.VMEM_SHARED((PADN,), jnp.float32)] * f
            + [pltpu.SemaphoreType.DMA, pltpu.SemaphoreType.DMA]
        ),
    )
    def prop(gt_hbm, src_hbm, dst_hbm, zeros_hbm, out_hbm, *scr):
        src_v, dst_v, colbuf, stage_v = scr[:4]
        gtab = scr[4:4 + f]
        a_sh = scr[4 + f:4 + 2 * f]
        psem, ssem = scr[4 + 2 * f:]
        c = lax.axis_index("c")
        s = lax.axis_index("s")
        wid = s * NC + c

        def col(st, jc):
            return colbuf.at[pl.ds((st * f + jc) * WIN, WIN)]

        cps = [pltpu.async_copy(gt_hbm.at[j], gtab[j], psem)
               for j in range(f)]
        for j in range(f):
            @pl.when(s == f + j)
            def _(j=j):
                pltpu.sync_copy(zeros_hbm, stage_v)
                pltpu.sync_copy(stage_v, a_sh[j])

        pltpu.sync_copy(src_hbm.at[wid], src_v)
        pltpu.sync_copy(dst_hbm.at[wid], dst_v)
        for h in cps:
            h.wait()
        plsc.subcore_barrier()

        def body(k, carry):
            for j in range(NSET):
                w = k * NSET + j
                nset = (j + 1) % NSET

                @pl.when(w >= NSET - 1)
                def _(w=w, nset=nset):
                    for jc in range(f):
                        pltpu.make_async_copy(
                            col(nset, jc),
                            a_sh[jc].at[dst_v.at[w - (NSET - 1)]], ssem).wait()

                for kk in range(WIN // 16):
                    idx = src_v[w, pl.ds(kk * 16, 16)]
                    for jc in range(f):
                        colbuf[pl.ds((j * f + jc) * WIN + kk * 16, 16)] = (
                            plsc.load_gather(gtab[jc], [idx]))
                dw = dst_v.at[w]
                for jc in range(f):
                    pltpu.async_copy(col(j, jc), a_sh[jc].at[dw], ssem,
                                     add=True)
            return carry

        lax.fori_loop(0, NWIN // NSET, body, 0)
        for w in range(NWIN - (NSET - 1), NWIN):
            for jc in range(f):
                pltpu.make_async_copy(
                    col(w % NSET, jc), a_sh[jc].at[dst_v.at[w]], ssem).wait()
        plsc.subcore_barrier()

        for j in range(f):
            @pl.when(s == j)
            def _(j=j):
                pltpu.sync_copy(a_sh[j].at[pl.ds(0, N)],
                                stage_v.at[pl.ds(0, N)])
                pltpu.sync_copy(stage_v.at[pl.ds(0, N)], out_hbm.at[c, j])

    return prop


_prop_kernel = _make_prop_kernel(F)
_prop_kernel2 = _make_prop_kernel(2)


def _tc_mm_body(x_ref, w1_ref, h0_ref):
    h0_ref[...] = lax.dot_general(w1_ref[...], x_ref[...],
                                  (((0,), (1,)), ((), ())),
                                  preferred_element_type=jnp.float32)


def _tc_a_body(degp_ref, x_ref, w1_ref, dinv_ref, g1_ref):
    deg = degp_ref[0:1, :] + degp_ref[1:2, :] + 1.0
    dinv = lax.rsqrt(deg)
    h0 = lax.dot_general(w1_ref[...], x_ref[...],
                         (((0,), (1,)), ((), ())),
                         preferred_element_type=jnp.float32)
    dinv_ref[...] = dinv
    g1_ref[...] = dinv * h0


def _tc_b_body(acc_ref, g_ref, dinv_ref, b_ref, w_ref, gnext_ref):
    st = acc_ref[0] + acc_ref[1] + g_ref[...]
    h = jnp.tanh(dinv_ref[...] * st + b_ref[...])
    gnext_ref[...] = dinv_ref[...] * lax.dot_general(
        w_ref[...], h, (((0,), (0,)), ((), ())),
        preferred_element_type=jnp.float32)


def _tc_c_body(acc_ref, g_ref, dinv_ref, b_ref, wc_ref, bc_ref,
               out_ref, h_ref):
    st = acc_ref[0] + acc_ref[1] + g_ref[...]
    h = jnp.tanh(dinv_ref[...] * st + b_ref[...])
    h_ref[...] = h
    out_ref[...] = lax.dot_general(
        wc_ref[...], h, (((0,), (0,)), ((), ())),
        preferred_element_type=jnp.float32) + bc_ref[...]


def _tc_mm(x, w1):
    return pl.pallas_call(
        _tc_mm_body,
        out_shape=jax.ShapeDtypeStruct((F, N), jnp.float32),
    )(x, w1)


def _tc_a(degp, x, w1):
    return pl.pallas_call(
        _tc_a_body,
        out_shape=(jax.ShapeDtypeStruct((1, N), jnp.float32),
                   jax.ShapeDtypeStruct((F, N), jnp.float32)),
    )(degp, x, w1)


def _tc_b(acc, g, dinv, b, w):
    return pl.pallas_call(
        _tc_b_body,
        out_shape=jax.ShapeDtypeStruct((w.shape[1], N), jnp.float32),
    )(acc, g, dinv, b, w)


def _tc_c(acc, g, dinv, b, wc, bc):
    return pl.pallas_call(
        _tc_c_body,
        out_shape=(jax.ShapeDtypeStruct((NUM_CLASSES, N), jnp.float32),
                   jax.ShapeDtypeStruct((2, N), jnp.float32)),
    )(acc, g, dinv, b, wc, bc)


def kernel(x, edge_index, W1, b1, W2, b2, W3, b3, Wc, bc):
    fill_src = jnp.broadcast_to((jnp.arange(PADE, dtype=jnp.int32) * 997) % N,
                                (NW, PADE))
    fill_dst = jnp.broadcast_to(
        N + (jnp.arange(PADE, dtype=jnp.int32) * 131) % DUMP, (NW, PADE))
    srcw = jnp.concatenate(
        [edge_index[0].reshape(NW, EPT), fill_src], axis=1).reshape(
            NW, NWIN, WIN)
    dstw = jnp.concatenate(
        [edge_index[1].reshape(NW, EPT), fill_dst], axis=1).reshape(
            NW, NWIN, WIN)
    z1 = jnp.zeros((PADN,), jnp.float32)

    b1c = b1.reshape(F, 1)
    b2c = b2.reshape(F, 1)
    b3c = b3.reshape(2, 1)
    bcc = bc.reshape(NUM_CLASSES, 1)

    degp = _deg_kernel(dstw, z1)
    dinv, g1 = _tc_a(degp, x, W1)
    acc1 = _prop_kernel(g1, srcw, dstw, z1)
    g2 = _tc_b(acc1, g1, dinv, b1c, W2)
    acc2 = _prop_kernel(g2, srcw, dstw, z1)
    g3 = _tc_b(acc2, g2, dinv, b2c, W3)
    acc3 = _prop_kernel2(g3, srcw, dstw, z1)
    out_t, h3_t = _tc_c(acc3, g3, dinv, b3c, Wc, bcc)
    return (jnp.transpose(out_t), jnp.transpose(h3_t))


# speedup vs baseline: 1.0170x; 1.0170x over previous
import functools

import jax
import jax.numpy as jnp
from jax import lax
from jax.experimental import pallas as pl
from jax.experimental.pallas import tpu as pltpu
from jax.experimental.pallas import tpu_sc as plsc

N = 10000
E = 320000
NUM_CLASSES = 4
F = 4
NC = 2
NS = 16
NW = NC * NS
EPT = E // NW
WIN = 128
NWIN = 80
EPTP = NWIN * WIN
PADE = EPTP - EPT
DUMP = 2048
PADN = N + DUMP
NSET = 5

_mesh = plsc.VectorSubcoreMesh(core_axis_name="c", subcore_axis_name="s")
_sc_params = pltpu.CompilerParams(use_tc_tiling_on_sc=False)
_sc_params_nl = pltpu.CompilerParams(use_tc_tiling_on_sc=False,
                                     needs_layout_passes=False)


@functools.partial(
    pl.kernel,
    out_type=jax.ShapeDtypeStruct((NC, N), jnp.float32),
    mesh=_mesh,
    compiler_params=_sc_params,
    scratch_types=[
        pltpu.VMEM((NWIN, WIN), jnp.int32),
        pltpu.VMEM((WIN,), jnp.float32),
        pltpu.VMEM((PADN,), jnp.float32),
        pltpu.VMEM_SHARED((PADN,), jnp.float32),
        pltpu.SemaphoreType.DMA,
    ],
)
def _deg_kernel(dst_hbm, zeros_hbm, out_hbm, dst_v, ones_v, stage_v, acc_sh,
                ssem):
    c = lax.axis_index("c")
    s = lax.axis_index("s")
    wid = s * NC + c

    @pl.when(s == 0)
    def _():
        pltpu.sync_copy(zeros_hbm, stage_v)
        pltpu.sync_copy(stage_v, acc_sh)

    for i in range(WIN // 16):
        ones_v[pl.ds(i * 16, 16)] = jnp.full((16,), 1.0, jnp.float32)
    pltpu.sync_copy(dst_hbm.at[wid], dst_v)
    plsc.subcore_barrier()

    def body(w, carry):
        pltpu.async_copy(ones_v, acc_sh.at[dst_v.at[w]], ssem, add=True)
        return carry

    lax.fori_loop(0, NWIN, body, 0)

    def drain(w, carry):
        pltpu.make_async_copy(ones_v, acc_sh.at[dst_v.at[w]], ssem).wait()
        return carry

    lax.fori_loop(0, NWIN, drain, 0)
    plsc.subcore_barrier()

    @pl.when(s == 0)
    def _():
        pltpu.sync_copy(acc_sh.at[pl.ds(0, N)], stage_v.at[pl.ds(0, N)])
        pltpu.sync_copy(stage_v.at[pl.ds(0, N)], out_hbm.at[c])


def _make_prop_kernel(f):
    @functools.partial(
        pl.kernel,
        out_type=jax.ShapeDtypeStruct((NC, f, N), jnp.float32),
        mesh=_mesh,
        compiler_params=_sc_params_nl,
        scratch_types=(
            [pltpu.VMEM((NWIN, WIN), jnp.int32),
             pltpu.VMEM((NWIN, WIN), jnp.int32),
             pltpu.VMEM((NSET * f * WIN,), jnp.float32),
             pltpu.VMEM((PADN,), jnp.float32)]
            + [pltpu.VMEM((N,), jnp.float32)] * f
            + [pltpu.VMEM_SHARED((PADN,), jnp.float32)] * f
            + [pltpu.SemaphoreType.DMA, pltpu.SemaphoreType.DMA]
        ),
    )
    def prop(gt_hbm, src_hbm, dst_hbm, zeros_hbm, out_hbm, *scr):
        src_v, dst_v, colbuf, stage_v = scr[:4]
        gtab = scr[4:4 + f]
        a_sh = scr[4 + f:4 + 2 * f]
        psem, ssem = scr[4 + 2 * f:]
        c = lax.axis_index("c")
        s = lax.axis_index("s")
        wid = s * NC + c

        def col(st, jc):
            return colbuf.at[pl.ds((st * f + jc) * WIN, WIN)]

        cps = [pltpu.async_copy(gt_hbm.at[j], gtab[j], psem)
               for j in range(f)]
        for j in range(f):
            @pl.when(s == f + j)
            def _(j=j):
                pltpu.sync_copy(zeros_hbm, stage_v)
                pltpu.sync_copy(stage_v, a_sh[j])

        pltpu.sync_copy(src_hbm.at[wid], src_v)
        pltpu.sync_copy(dst_hbm.at[wid], dst_v)
        for h in cps:
            h.wait()
        plsc.subcore_barrier()

        def body(k, carry):
            for j in range(NSET):
                w = k * NSET + j
                nset = (j + 1) % NSET

                @pl.when(w >= NSET - 1)
                def _(w=w, nset=nset):
                    for jc in range(f):
                        pltpu.make_async_copy(
                            col(nset, jc),
                            a_sh[jc].at[dst_v.at[w - (NSET - 1)]], ssem).wait()

                for kk in range(WIN // 16):
                    idx = src_v[w, pl.ds(kk * 16, 16)]
                    for jc in range(f):
                        colbuf[pl.ds((j * f + jc) * WIN + kk * 16, 16)] = (
                            plsc.load_gather(gtab[jc], [idx]))
                dw = dst_v.at[w]
                for jc in range(f):
                    pltpu.async_copy(col(j, jc), a_sh[jc].at[dw], ssem,
                                     add=True)
            return carry

        lax.fori_loop(0, NWIN // NSET, body, 0)
        for w in range(NWIN - (NSET - 1), NWIN):
            for jc in range(f):
                pltpu.make_async_copy(
                    col(w % NSET, jc), a_sh[jc].at[dst_v.at[w]], ssem).wait()
        plsc.subcore_barrier()

        for j in range(f):
            @pl.when(s == j)
            def _(j=j):
                pltpu.sync_copy(a_sh[j].at[pl.ds(0, N)],
                                stage_v.at[pl.ds(0, N)])
                pltpu.sync_copy(stage_v.at[pl.ds(0, N)], out_hbm.at[c, j])

    return prop


_prop_kernel = _make_prop_kernel(F)
_prop_kernel2 = _make_prop_kernel(2)


def _tc_mm_body(x_ref, w1_ref, h0_ref):
    h0_ref[...] = lax.dot_general(w1_ref[...], x_ref[...],
                                  (((0,), (1,)), ((), ())),
                                  preferred_element_type=jnp.float32)


def _tc_a_body(degp_ref, h0_ref, dinv_ref, g1_ref):
    deg = degp_ref[0:1, :] + degp_ref[1:2, :] + 1.0
    dinv = lax.rsqrt(deg)
    dinv_ref[...] = dinv
    g1_ref[...] = dinv * h0_ref[...]


def _tc_b_body(acc_ref, g_ref, dinv_ref, b_ref, w_ref, gnext_ref):
    st = acc_ref[0] + acc_ref[1] + g_ref[...]
    h = jnp.tanh(dinv_ref[...] * st + b_ref[...])
    gnext_ref[...] = dinv_ref[...] * lax.dot_general(
        w_ref[...], h, (((0,), (0,)), ((), ())),
        preferred_element_type=jnp.float32)


def _tc_c_body(acc_ref, g_ref, dinv_ref, b_ref, wc_ref, bc_ref,
               out_ref, h_ref):
    st = acc_ref[0] + acc_ref[1] + g_ref[...]
    h = jnp.tanh(dinv_ref[...] * st + b_ref[...])
    h_ref[...] = h
    out_ref[...] = lax.dot_general(
        wc_ref[...], h, (((0,), (0,)), ((), ())),
        preferred_element_type=jnp.float32) + bc_ref[...]


def _tc_mm(x, w1):
    return pl.pallas_call(
        _tc_mm_body,
        out_shape=jax.ShapeDtypeStruct((F, N), jnp.float32),
    )(x, w1)


def _tc_a(degp, h0):
    return pl.pallas_call(
        _tc_a_body,
        out_shape=(jax.ShapeDtypeStruct((1, N), jnp.float32),
                   jax.ShapeDtypeStruct((F, N), jnp.float32)),
    )(degp, h0)


def _tc_b(acc, g, dinv, b, w):
    return pl.pallas_call(
        _tc_b_body,
        out_shape=jax.ShapeDtypeStruct((w.shape[1], N), jnp.float32),
    )(acc, g, dinv, b, w)


def _tc_c(acc, g, dinv, b, wc, bc):
    return pl.pallas_call(
        _tc_c_body,
        out_shape=(jax.ShapeDtypeStruct((NUM_CLASSES, N), jnp.float32),
                   jax.ShapeDtypeStruct((2, N), jnp.float32)),
    )(acc, g, dinv, b, wc, bc)


def kernel(x, edge_index, W1, b1, W2, b2, W3, b3, Wc, bc):
    fill_src = jnp.broadcast_to((jnp.arange(PADE, dtype=jnp.int32) * 997) % N,
                                (NW, PADE))
    fill_dst = jnp.broadcast_to(
        N + (jnp.arange(PADE, dtype=jnp.int32) * 131) % DUMP, (NW, PADE))
    srcw = jnp.concatenate(
        [edge_index[0].reshape(NW, EPT), fill_src], axis=1).reshape(
            NW, NWIN, WIN)
    dstw = jnp.concatenate(
        [edge_index[1].reshape(NW, EPT), fill_dst], axis=1).reshape(
            NW, NWIN, WIN)
    z1 = jnp.zeros((PADN,), jnp.float32)

    b1c = b1.reshape(F, 1)
    b2c = b2.reshape(F, 1)
    b3c = b3.reshape(2, 1)
    bcc = bc.reshape(NUM_CLASSES, 1)

    degp = _deg_kernel(dstw, z1)
    h0 = _tc_mm(x, W1)
    dinv, g1 = _tc_a(degp, h0)
    acc1 = _prop_kernel(g1, srcw, dstw, z1)
    g2 = _tc_b(acc1, g1, dinv, b1c, W2)
    acc2 = _prop_kernel(g2, srcw, dstw, z1)
    g3 = _tc_b(acc2, g2, dinv, b2c, W3)
    acc3 = _prop_kernel2(g3, srcw, dstw, z1)
    out_t, h3_t = _tc_c(acc3, g3, dinv, b3c, Wc, bcc)
    return (jnp.transpose(out_t), jnp.transpose(h3_t))
